# baseline (speedup 1.0000x reference)
import functools
import math

import jax
import jax.numpy as jnp
from jax import lax
from jax.experimental import pallas as pl
from jax.experimental.pallas import tpu as pltpu
from jax.experimental.pallas import tpu_sc as plsc

_D = 64
_SCALE = math.sqrt(_D)
_NW = 32
_CHUNK = 128
_LANES = 16


def _emb_body(x_hbm, lut_hbm, out_hbm, idx_v, rows_v, sem):
    n_chunks = x_hbm.shape[1]
    wid = lax.axis_index("s") * 2 + lax.axis_index("c")
    base = wid * (n_chunks * _CHUNK)

    pltpu.sync_copy(x_hbm.at[wid], idx_v)

    @pl.loop(0, n_chunks)
    def _chunk(g):
        pltpu.async_copy(lut_hbm.at[idx_v.at[g]], rows_v, sem).wait()

        @pl.loop(0, _CHUNK)
        def _row(i):
            for j in range(_D // _LANES):
                sl = pl.ds(j * _LANES, _LANES)
                rows_v[i, sl] = rows_v[i, sl] * _SCALE

        pltpu.sync_copy(rows_v, out_hbm.at[pl.ds(base + g * _CHUNK, _CHUNK)])


def kernel(x, lut):
    b, s = x.shape
    total = b * s
    n_chunks = total // (_NW * _CHUNK)
    x_grid = x.reshape(_NW, n_chunks, _CHUNK)

    mesh = plsc.VectorSubcoreMesh(core_axis_name="c", subcore_axis_name="s")
    run = functools.partial(
        pl.kernel,
        out_type=jax.ShapeDtypeStruct((total, _D), jnp.float32),
        mesh=mesh,
        scratch_types=[
            pltpu.VMEM((n_chunks, _CHUNK), jnp.int32),
            pltpu.VMEM((_CHUNK, _D), jnp.float32),
            pltpu.SemaphoreType.DMA,
        ],
        compiler_params=pltpu.CompilerParams(use_tc_tiling_on_sc=False),
    )(_emb_body)
    out = run(x_grid, lut)
    return out.reshape(b, s, _D)


# speedup vs baseline: 1.0725x; 1.0725x over previous
import functools
import math

import jax
import jax.numpy as jnp
from jax import lax
from jax.experimental import pallas as pl
from jax.experimental.pallas import tpu as pltpu
from jax.experimental.pallas import tpu_sc as plsc

_D = 64
_SCALE = math.sqrt(_D)
_NW = 32
_CHUNK = 128
_LANES = 16
_NBUF = 5
_LEAD = 2


def _emb_body(x_hbm, lut_hbm, out_hbm, idx_v, rows_v, sem_in, sem_out):
    n_chunks = x_hbm.shape[1]
    wid = lax.axis_index("s") * 2 + lax.axis_index("c")
    base = wid * (n_chunks * _CHUNK)

    pltpu.sync_copy(x_hbm.at[wid], idx_v)

    def start_gather(g, b):
        pltpu.async_copy(lut_hbm.at[idx_v.at[g]], rows_v.at[b], sem_in)

    def wait_gather(b):
        pltpu.make_async_copy(
            lut_hbm.at[pl.ds(0, _CHUNK)], rows_v.at[b], sem_in
        ).wait()

    def start_store(g, b):
        pltpu.async_copy(
            rows_v.at[b], out_hbm.at[pl.ds(base + g * _CHUNK, _CHUNK)], sem_out
        )

    def wait_store(b):
        pltpu.make_async_copy(
            rows_v.at[b], out_hbm.at[pl.ds(base, _CHUNK)], sem_out
        ).wait()

    for g in range(_LEAD):
        start_gather(g, g % _NBUF)

    @pl.loop(0, n_chunks, step=_NBUF)
    def _outer(g0):
        for b in range(_NBUF):
            g = g0 + b
            wait_gather(b)

            @pl.loop(0, _CHUNK, unroll=8)
            def _row(i):
                for j in range(_D // _LANES):
                    sl = pl.ds(j * _LANES, _LANES)
                    rows_v[b, i, sl] = rows_v[b, i, sl] * _SCALE

            start_store(g, b)

            h = g + _LEAD
            hb = (b + _LEAD) % _NBUF

            @pl.when(h < n_chunks)
            def _prefetch():
                @pl.when(h >= _NBUF)
                def _drain_prior_store():
                    wait_store(hb)

                start_gather(h, hb)

    for b in range(_NBUF):
        wait_store(b)


def kernel(x, lut):
    b, s = x.shape
    total = b * s
    n_chunks = total // (_NW * _CHUNK)
    x_grid = x.reshape(_NW, n_chunks, _CHUNK)

    mesh = plsc.VectorSubcoreMesh(core_axis_name="c", subcore_axis_name="s")
    run = functools.partial(
        pl.kernel,
        out_type=jax.ShapeDtypeStruct((total, _D), jnp.float32),
        mesh=mesh,
        scratch_types=[
            pltpu.VMEM((n_chunks, _CHUNK), jnp.int32),
            pltpu.VMEM((_NBUF, _CHUNK, _D), jnp.float32),
            pltpu.SemaphoreType.DMA,
            pltpu.SemaphoreType.DMA,
        ],
        compiler_params=pltpu.CompilerParams(use_tc_tiling_on_sc=False),
    )(_emb_body)
    out = run(x_grid, lut)
    return out.reshape(b, s, _D)
